# baseline (speedup 1.0000x reference)
import functools

import jax
import jax.numpy as jnp
from jax import lax
from jax.experimental import pallas as pl
from jax.experimental.pallas import tpu as pltpu
from jax.experimental.pallas import tpu_sc as plsc

N_ROWS = 320000
D = 128
N_SEG = 10000
NC = 2
NS = 16
NW = NC * NS
UNIT = 128
N_UNITS = N_ROWS // UNIT
BASE = N_UNITS // NW
EXTRA = N_UNITS % NW
SEG_STRIDE = 624
SEG_COPY = 640


def _sc_partial(x, batch):
    @functools.partial(
        pl.kernel,
        out_type=jax.ShapeDtypeStruct((NC, N_SEG, D), jnp.float32),
        mesh=plsc.VectorSubcoreMesh(core_axis_name="c", subcore_axis_name="s"),
        scratch_types=[
            pltpu.VMEM_SHARED((N_SEG, D), jnp.float32),
            pltpu.VMEM((UNIT, D), jnp.float32),
            pltpu.VMEM((UNIT, D), jnp.float32),
            pltpu.VMEM((BASE + 1, 1, UNIT), jnp.int32),
            pltpu.SemaphoreType.DMA,
            pltpu.SemaphoreType.DMA,
            pltpu.SemaphoreType.DMA,
            pltpu.SemaphoreType.DMA,
            pltpu.SemaphoreType.DMA,
        ],
    )
    def run(x_hbm, b3_hbm, out_hbm, acc,
            xb0, xb1, itab,
            sem0, sem1, isem, ssem0, ssem1):
        c = lax.axis_index("c")
        s = lax.axis_index("s")
        w = c * NS + s

        @pl.loop(0, UNIT)
        def _(i):
            @pl.loop(0, D, step=16)
            def _(j):
                xb0[i, pl.ds(j, 16)] = jnp.zeros((16,), jnp.float32)

        seg0 = s * SEG_STRIDE
        zcp = [
            pltpu.async_copy(xb0, acc.at[pl.ds(seg0 + t * UNIT, UNIT)], sem0)
            for t in range(SEG_COPY // UNIT)
        ]
        for cp in zcp:
            cp.wait()
        plsc.subcore_barrier()

        cnt = jnp.where(w < EXTRA, BASE + 1, BASE)
        u0 = w * BASE + jnp.minimum(w, EXTRA)

        icp = pltpu.async_copy(b3_hbm.at[pl.ds(u0, BASE)],
                               itab.at[pl.ds(0, BASE)], isem)

        def issue(j, xb, sem):
            pltpu.async_copy(x_hbm.at[pl.ds((u0 + j) * UNIT, UNIT)], xb, sem)

        def wait_load(xb, sem):
            pltpu.make_async_copy(x_hbm.at[pl.ds(0, UNIT)], xb, sem).wait()

        def scatter_start(j, xb, ssem):
            pltpu.async_copy(xb, acc.at[itab.at[j, 0]], ssem, add=True)

        def scatter_wait(j, xb, ssem):
            pltpu.make_async_copy(xb, acc.at[itab.at[j, 0]], ssem).wait()

        issue(0, xb0, sem0)
        issue(1, xb1, sem1)
        icp.wait()

        @pl.when(w < EXTRA)
        def _():
            pltpu.async_copy(b3_hbm.at[pl.ds(u0 + BASE, 1)],
                             itab.at[pl.ds(BASE, 1)], isem).wait()

        @pl.loop(0, cnt // 2)
        def _(p):
            j0 = 2 * p
            j1 = 2 * p + 1
            wait_load(xb0, sem0)
            scatter_start(j0, xb0, ssem0)
            wait_load(xb1, sem1)
            scatter_start(j1, xb1, ssem1)
            scatter_wait(j0, xb0, ssem0)

            @pl.when(j1 + 1 < cnt)
            def _():
                issue(j1 + 1, xb0, sem0)

            scatter_wait(j1, xb1, ssem1)

            @pl.when(j1 + 2 < cnt)
            def _():
                issue(j1 + 2, xb1, sem1)

        @pl.when(cnt % 2 == 1)
        def _():
            wait_load(xb0, sem0)
            scatter_start(cnt - 1, xb0, ssem0)
            scatter_wait(cnt - 1, xb0, ssem0)

        plsc.subcore_barrier()

        pltpu.sync_copy(acc.at[pl.ds(seg0, SEG_COPY)],
                        out_hbm.at[c, pl.ds(seg0, SEG_COPY)])

    return run(x, batch)


def _combine_body(p_ref, o_ref):
    o_ref[...] = p_ref[0] + p_ref[1]


def _tc_combine(partial):
    blk = 2000
    return pl.pallas_call(
        _combine_body,
        grid=(N_SEG // blk,),
        in_specs=[pl.BlockSpec((NC, blk, D), lambda i: (0, i, 0))],
        out_specs=pl.BlockSpec((blk, D), lambda i: (i, 0)),
        out_shape=jax.ShapeDtypeStruct((N_SEG, D), jnp.float32),
    )(partial)


def kernel(x, batch):
    batch3d = batch.astype(jnp.int32).reshape(N_UNITS, 1, UNIT)
    partial = _sc_partial(x, batch3d)
    return _tc_combine(partial)


# speedup vs baseline: 1.0011x; 1.0011x over previous
import functools

import jax
import jax.numpy as jnp
from jax import lax
from jax.experimental import pallas as pl
from jax.experimental.pallas import tpu as pltpu
from jax.experimental.pallas import tpu_sc as plsc

N_ROWS = 320000
D = 128
N_SEG = 10000
NC = 2
NS = 16
NW = NC * NS
UNIT = 128
N_UNITS = N_ROWS // UNIT
BASE = N_UNITS // NW
EXTRA = N_UNITS % NW
SEG_STRIDE = 624
SEG_COPY = 640


def _sc_partial(x, batch):
    @functools.partial(
        pl.kernel,
        out_type=jax.ShapeDtypeStruct((NC, N_SEG, D), jnp.float32),
        mesh=plsc.VectorSubcoreMesh(core_axis_name="c", subcore_axis_name="s"),
        scratch_types=[
            pltpu.VMEM_SHARED((N_SEG, D), jnp.float32),
            pltpu.VMEM((UNIT, D), jnp.float32),
            pltpu.VMEM((UNIT, D), jnp.float32),
            pltpu.VMEM((BASE + 1, 1, UNIT), jnp.int32),
            pltpu.VMEM((UNIT,), jnp.int32),
            pltpu.VMEM((UNIT,), jnp.int32),
            pltpu.SemaphoreType.DMA,
            pltpu.SemaphoreType.DMA,
            pltpu.SemaphoreType.DMA,
            pltpu.SemaphoreType.DMA,
            pltpu.SemaphoreType.DMA,
        ],
    )
    def run(x_hbm, b3_hbm, out_hbm, acc,
            xb0, xb1, itab, ib0, ib1,
            sem0, sem1, isem, ssem0, ssem1):
        c = lax.axis_index("c")
        s = lax.axis_index("s")
        w = c * NS + s

        @pl.loop(0, UNIT)
        def _(i):
            @pl.loop(0, D, step=16)
            def _(j):
                xb0[i, pl.ds(j, 16)] = jnp.zeros((16,), jnp.float32)

        seg0 = s * SEG_STRIDE
        zcp = [
            pltpu.async_copy(xb0, acc.at[pl.ds(seg0 + t * UNIT, UNIT)], sem0)
            for t in range(SEG_COPY // UNIT)
        ]
        for cp in zcp:
            cp.wait()
        plsc.subcore_barrier()

        cnt = jnp.where(w < EXTRA, BASE + 1, BASE)
        u0 = w * BASE + jnp.minimum(w, EXTRA)

        icp = pltpu.async_copy(b3_hbm.at[pl.ds(u0, BASE)],
                               itab.at[pl.ds(0, BASE)], isem)

        def issue(j, xb, sem):
            pltpu.async_copy(x_hbm.at[pl.ds((u0 + j) * UNIT, UNIT)], xb, sem)

        def wait_load(xb, sem):
            pltpu.make_async_copy(x_hbm.at[pl.ds(0, UNIT)], xb, sem).wait()

        def copy_ids(j, ib):
            for k in range(0, UNIT, 16):
                ib[pl.ds(k, 16)] = itab[j, 0, pl.ds(k, 16)]

        def scatter_start(xb, ib, ssem):
            pltpu.async_copy(xb, acc.at[ib], ssem, add=True)

        def scatter_wait(xb, ib, ssem):
            pltpu.make_async_copy(xb, acc.at[ib], ssem).wait()

        issue(0, xb0, sem0)
        issue(1, xb1, sem1)
        icp.wait()

        @pl.when(w < EXTRA)
        def _():
            pltpu.async_copy(b3_hbm.at[pl.ds(u0 + BASE, 1)],
                             itab.at[pl.ds(BASE, 1)], isem).wait()

        @pl.loop(0, cnt // 2)
        def _(p):
            j1 = 2 * p + 1
            wait_load(xb0, sem0)
            copy_ids(j1 - 1, ib0)
            scatter_start(xb0, ib0, ssem0)
            wait_load(xb1, sem1)
            copy_ids(j1, ib1)
            scatter_start(xb1, ib1, ssem1)
            scatter_wait(xb0, ib0, ssem0)

            @pl.when(j1 + 1 < cnt)
            def _():
                issue(j1 + 1, xb0, sem0)

            scatter_wait(xb1, ib1, ssem1)

            @pl.when(j1 + 2 < cnt)
            def _():
                issue(j1 + 2, xb1, sem1)

        @pl.when(cnt % 2 == 1)
        def _():
            wait_load(xb0, sem0)
            copy_ids(cnt - 1, ib0)
            scatter_start(xb0, ib0, ssem0)
            scatter_wait(xb0, ib0, ssem0)

        plsc.subcore_barrier()

        pltpu.sync_copy(acc.at[pl.ds(seg0, SEG_COPY)],
                        out_hbm.at[c, pl.ds(seg0, SEG_COPY)])

    return run(x, batch)


def _combine_body(p_ref, o_ref):
    o_ref[...] = p_ref[0] + p_ref[1]


def _tc_combine(partial):
    blk = 2000
    return pl.pallas_call(
        _combine_body,
        grid=(N_SEG // blk,),
        in_specs=[pl.BlockSpec((NC, blk, D), lambda i: (0, i, 0))],
        out_specs=pl.BlockSpec((blk, D), lambda i: (i, 0)),
        out_shape=jax.ShapeDtypeStruct((N_SEG, D), jnp.float32),
    )(partial)


def kernel(x, batch):
    batch3d = batch.astype(jnp.int32).reshape(N_UNITS, 1, UNIT)
    partial = _sc_partial(x, batch3d)
    return _tc_combine(partial)


# speedup vs baseline: 1.2076x; 1.2063x over previous
import functools

import jax
import jax.numpy as jnp
from jax import lax
from jax.experimental import pallas as pl
from jax.experimental.pallas import tpu as pltpu
from jax.experimental.pallas import tpu_sc as plsc

N_ROWS = 320000
D = 128
N_SEG = 10000
NC = 2
NS = 16
NW = NC * NS
UNIT = 80
DEPTH = 3
N_UNITS = N_ROWS // UNIT
BASE = N_UNITS // NW
EXTRA = N_UNITS % NW
SEG_STRIDE = 624
SEG_COPY = 640


def _sc_partial(x, batch):
    @functools.partial(
        pl.kernel,
        out_type=jax.ShapeDtypeStruct((NC, N_SEG, D), jnp.float32),
        mesh=plsc.VectorSubcoreMesh(core_axis_name="c", subcore_axis_name="s"),
        scratch_types=(
            [pltpu.VMEM_SHARED((N_SEG, D), jnp.float32)]
            + [pltpu.VMEM((UNIT, D), jnp.float32)] * DEPTH
            + [pltpu.VMEM((BASE + 1, 1, UNIT), jnp.int32)]
            + [pltpu.VMEM((UNIT,), jnp.int32)] * DEPTH
            + [pltpu.SemaphoreType.DMA] * DEPTH
            + [pltpu.SemaphoreType.DMA]
            + [pltpu.SemaphoreType.DMA] * DEPTH
        ),
    )
    def run(x_hbm, b3_hbm, out_hbm, acc, *rest):
        xbs = rest[:DEPTH]
        itab = rest[DEPTH]
        ibs = rest[DEPTH + 1:2 * DEPTH + 1]
        lsems = rest[2 * DEPTH + 1:3 * DEPTH + 1]
        isem = rest[3 * DEPTH + 1]
        ssems = rest[3 * DEPTH + 2:]
        xb0, sem0 = xbs[0], lsems[0]
        c = lax.axis_index("c")
        s = lax.axis_index("s")
        w = c * NS + s

        @pl.loop(0, UNIT)
        def _(i):
            @pl.loop(0, D, step=16)
            def _(j):
                xb0[i, pl.ds(j, 16)] = jnp.zeros((16,), jnp.float32)

        seg0 = s * SEG_STRIDE
        zcp = [
            pltpu.async_copy(xb0, acc.at[pl.ds(seg0 + t * UNIT, UNIT)], sem0)
            for t in range(SEG_COPY // UNIT)
        ]
        for cp in zcp:
            cp.wait()
        plsc.subcore_barrier()

        cnt = jnp.where(w < EXTRA, BASE + 1, BASE)
        u0 = w * BASE + jnp.minimum(w, EXTRA)

        icp = pltpu.async_copy(b3_hbm.at[pl.ds(u0, BASE)],
                               itab.at[pl.ds(0, BASE)], isem)

        def issue(j, xb, sem):
            pltpu.async_copy(x_hbm.at[pl.ds((u0 + j) * UNIT, UNIT)], xb, sem)

        def wait_load(xb, sem):
            pltpu.make_async_copy(x_hbm.at[pl.ds(0, UNIT)], xb, sem).wait()

        def copy_ids(j, ib):
            for k in range(0, UNIT, 16):
                ib[pl.ds(k, 16)] = itab[j, 0, pl.ds(k, 16)]

        def scatter_start(xb, ib, ssem):
            pltpu.async_copy(xb, acc.at[ib], ssem, add=True)

        def scatter_wait(xb, ib, ssem):
            pltpu.make_async_copy(xb, acc.at[ib], ssem).wait()

        for k in range(DEPTH):
            issue(k, xbs[k], lsems[k])
        icp.wait()

        @pl.when(w < EXTRA)
        def _():
            pltpu.async_copy(b3_hbm.at[pl.ds(u0 + BASE, 1)],
                             itab.at[pl.ds(BASE, 1)], isem).wait()

        @pl.loop(0, cnt // DEPTH)
        def _(p):
            j0 = DEPTH * p
            for k in range(DEPTH):
                wait_load(xbs[k], lsems[k])
                copy_ids(j0 + k, ibs[k])
                scatter_start(xbs[k], ibs[k], ssems[k])
            for k in range(DEPTH):
                scatter_wait(xbs[k], ibs[k], ssems[k])

                def _reissue(k=k):
                    issue(j0 + DEPTH + k, xbs[k], lsems[k])

                pl.when(j0 + DEPTH + k < cnt)(_reissue)

        t0 = (cnt // DEPTH) * DEPTH
        for k in range(DEPTH - 1):

            def _tail(k=k):
                wait_load(xbs[k], lsems[k])
                copy_ids(t0 + k, ibs[k])
                scatter_start(xbs[k], ibs[k], ssems[k])

            pl.when(t0 + k < cnt)(_tail)
        for k in range(DEPTH - 1):

            def _tail_wait(k=k):
                scatter_wait(xbs[k], ibs[k], ssems[k])

            pl.when(t0 + k < cnt)(_tail_wait)

        plsc.subcore_barrier()

        pltpu.sync_copy(acc.at[pl.ds(seg0, SEG_COPY)],
                        out_hbm.at[c, pl.ds(seg0, SEG_COPY)])

    return run(x, batch)


def _combine_body(p_ref, o_ref):
    o_ref[...] = p_ref[0] + p_ref[1]


def _tc_combine(partial):
    blk = 2000
    return pl.pallas_call(
        _combine_body,
        grid=(N_SEG // blk,),
        in_specs=[pl.BlockSpec((NC, blk, D), lambda i: (0, i, 0))],
        out_specs=pl.BlockSpec((blk, D), lambda i: (i, 0)),
        out_shape=jax.ShapeDtypeStruct((N_SEG, D), jnp.float32),
    )(partial)


def kernel(x, batch):
    batch3d = batch.astype(jnp.int32).reshape(N_UNITS, 1, UNIT)
    partial = _sc_partial(x, batch3d)
    return _tc_combine(partial)


# speedup vs baseline: 1.2534x; 1.0379x over previous
import functools

import jax
import jax.numpy as jnp
from jax import lax
from jax.experimental import pallas as pl
from jax.experimental.pallas import tpu as pltpu
from jax.experimental.pallas import tpu_sc as plsc

N_ROWS = 320000
D = 128
N_SEG = 10000
NC = 2
NS = 16
NW = NC * NS
UNIT = 64
DEPTH = 4
N_UNITS = N_ROWS // UNIT
BASE = N_UNITS // NW
EXTRA = (N_UNITS % NW) // 2
SEG_STRIDE = 624
SEG_COPY = 640


def _sc_partial(x, batch):
    @functools.partial(
        pl.kernel,
        out_type=jax.ShapeDtypeStruct((NC, N_SEG, D), jnp.float32),
        mesh=plsc.VectorSubcoreMesh(core_axis_name="c", subcore_axis_name="s"),
        scratch_types=(
            [pltpu.VMEM_SHARED((N_SEG, D), jnp.float32)]
            + [pltpu.VMEM((UNIT, D), jnp.float32)] * DEPTH
            + [pltpu.VMEM((BASE // 2 + 1, 1, 128), jnp.int32)]
            + [pltpu.VMEM((UNIT,), jnp.int32)] * DEPTH
            + [pltpu.SemaphoreType.DMA] * DEPTH
            + [pltpu.SemaphoreType.DMA]
            + [pltpu.SemaphoreType.DMA] * DEPTH
        ),
    )
    def run(x_hbm, b3_hbm, out_hbm, acc, *rest):
        xbs = rest[:DEPTH]
        itab = rest[DEPTH]
        ibs = rest[DEPTH + 1:2 * DEPTH + 1]
        lsems = rest[2 * DEPTH + 1:3 * DEPTH + 1]
        isem = rest[3 * DEPTH + 1]
        ssems = rest[3 * DEPTH + 2:]
        xb0, sem0 = xbs[0], lsems[0]
        c = lax.axis_index("c")
        s = lax.axis_index("s")
        w = c * NS + s

        @pl.loop(0, UNIT)
        def _(i):
            @pl.loop(0, D, step=16)
            def _(j):
                xb0[i, pl.ds(j, 16)] = jnp.zeros((16,), jnp.float32)

        seg0 = s * SEG_STRIDE
        zcp = [
            pltpu.async_copy(xb0, acc.at[pl.ds(seg0 + t * UNIT, UNIT)], sem0)
            for t in range(SEG_COPY // UNIT)
        ]
        for cp in zcp:
            cp.wait()
        plsc.subcore_barrier()

        cnt = jnp.where(w < EXTRA, BASE + 2, BASE)
        u0 = w * BASE + 2 * jnp.minimum(w, EXTRA)

        icp = pltpu.async_copy(b3_hbm.at[pl.ds(u0 // 2, BASE // 2)],
                               itab.at[pl.ds(0, BASE // 2)], isem)

        def issue(j, xb, sem):
            pltpu.async_copy(x_hbm.at[pl.ds((u0 + j) * UNIT, UNIT)], xb, sem)

        def wait_load(xb, sem):
            pltpu.make_async_copy(x_hbm.at[pl.ds(0, UNIT)], xb, sem).wait()

        def copy_ids(j, ib, off):
            for k in range(0, UNIT, 16):
                ib[pl.ds(k, 16)] = itab[j // 2, 0, pl.ds(off + k, 16)]

        def scatter_start(xb, ib, ssem):
            pltpu.async_copy(xb, acc.at[ib], ssem, add=True)

        def scatter_wait(xb, ib, ssem):
            pltpu.make_async_copy(xb, acc.at[ib], ssem).wait()

        for k in range(DEPTH):
            issue(k, xbs[k], lsems[k])
        icp.wait()

        @pl.when(w < EXTRA)
        def _():
            pltpu.async_copy(b3_hbm.at[pl.ds(u0 // 2 + BASE // 2, 1)],
                             itab.at[pl.ds(BASE // 2, 1)], isem).wait()

        @pl.loop(0, cnt // DEPTH)
        def _(p):
            j0 = DEPTH * p
            for k in range(DEPTH):
                wait_load(xbs[k], lsems[k])
                copy_ids(j0 + k, ibs[k], (k % 2) * UNIT)
                scatter_start(xbs[k], ibs[k], ssems[k])
            for k in range(DEPTH):
                scatter_wait(xbs[k], ibs[k], ssems[k])

                def _reissue(k=k):
                    issue(j0 + DEPTH + k, xbs[k], lsems[k])

                pl.when(j0 + DEPTH + k < cnt)(_reissue)

        t0 = (cnt // DEPTH) * DEPTH
        for k in range(DEPTH - 1):

            def _tail(k=k):
                wait_load(xbs[k], lsems[k])
                copy_ids(t0 + k, ibs[k], (k % 2) * UNIT)
                scatter_start(xbs[k], ibs[k], ssems[k])

            pl.when(t0 + k < cnt)(_tail)
        for k in range(DEPTH - 1):

            def _tail_wait(k=k):
                scatter_wait(xbs[k], ibs[k], ssems[k])

            pl.when(t0 + k < cnt)(_tail_wait)

        plsc.subcore_barrier()

        pltpu.sync_copy(acc.at[pl.ds(seg0, SEG_COPY)],
                        out_hbm.at[c, pl.ds(seg0, SEG_COPY)])

    return run(x, batch)


def _combine_body(p_ref, o_ref):
    o_ref[...] = p_ref[0] + p_ref[1]


def _tc_combine(partial):
    blk = 2000
    return pl.pallas_call(
        _combine_body,
        grid=(N_SEG // blk,),
        in_specs=[pl.BlockSpec((NC, blk, D), lambda i: (0, i, 0))],
        out_specs=pl.BlockSpec((blk, D), lambda i: (i, 0)),
        out_shape=jax.ShapeDtypeStruct((N_SEG, D), jnp.float32),
    )(partial)


def kernel(x, batch):
    batch3d = batch.astype(jnp.int32).reshape(N_UNITS // 2, 1, 2 * UNIT)
    partial = _sc_partial(x, batch3d)
    return _tc_combine(partial)


# speedup vs baseline: 1.2738x; 1.0163x over previous
import functools

import jax
import jax.numpy as jnp
from jax import lax
from jax.experimental import pallas as pl
from jax.experimental.pallas import tpu as pltpu
from jax.experimental.pallas import tpu_sc as plsc

N_ROWS = 320000
D = 128
N_SEG = 10000
NC = 2
NS = 16
NW = NC * NS
UNIT = 32
DEPTH = 8
PACK = 128 // UNIT
N_UNITS = N_ROWS // UNIT
BASE = N_UNITS // NW
EXTRA = (N_UNITS % NW) // PACK
SEG_STRIDE = 624
SEG_COPY = 640


def _sc_partial(x, batch):
    @functools.partial(
        pl.kernel,
        out_type=jax.ShapeDtypeStruct((NC, N_SEG, D), jnp.float32),
        mesh=plsc.VectorSubcoreMesh(core_axis_name="c", subcore_axis_name="s"),
        scratch_types=(
            [pltpu.VMEM_SHARED((N_SEG, D), jnp.float32)]
            + [pltpu.VMEM((UNIT, D), jnp.float32)] * DEPTH
            + [pltpu.VMEM((BASE // PACK + 1, 1, 128), jnp.int32)]
            + [pltpu.VMEM((UNIT,), jnp.int32)] * DEPTH
            + [pltpu.SemaphoreType.DMA] * DEPTH
            + [pltpu.SemaphoreType.DMA]
            + [pltpu.SemaphoreType.DMA] * DEPTH
        ),
    )
    def run(x_hbm, b3_hbm, out_hbm, acc, *rest):
        xbs = rest[:DEPTH]
        itab = rest[DEPTH]
        ibs = rest[DEPTH + 1:2 * DEPTH + 1]
        lsems = rest[2 * DEPTH + 1:3 * DEPTH + 1]
        isem = rest[3 * DEPTH + 1]
        ssems = rest[3 * DEPTH + 2:]
        xb0, sem0 = xbs[0], lsems[0]
        c = lax.axis_index("c")
        s = lax.axis_index("s")
        w = c * NS + s

        @pl.loop(0, UNIT)
        def _(i):
            @pl.loop(0, D, step=16)
            def _(j):
                xb0[i, pl.ds(j, 16)] = jnp.zeros((16,), jnp.float32)

        seg0 = s * SEG_STRIDE
        zcp = [
            pltpu.async_copy(xb0, acc.at[pl.ds(seg0 + t * UNIT, UNIT)], sem0)
            for t in range(SEG_COPY // UNIT)
        ]
        for cp in zcp:
            cp.wait()
        plsc.subcore_barrier()

        cnt = jnp.where(w < EXTRA, BASE + PACK, BASE)
        u0 = w * BASE + PACK * jnp.minimum(w, EXTRA)

        icp = pltpu.async_copy(b3_hbm.at[pl.ds(u0 // PACK, BASE // PACK)],
                               itab.at[pl.ds(0, BASE // PACK)], isem)

        def issue(j, xb, sem):
            pltpu.async_copy(x_hbm.at[pl.ds((u0 + j) * UNIT, UNIT)], xb, sem)

        def wait_load(xb, sem):
            pltpu.make_async_copy(x_hbm.at[pl.ds(0, UNIT)], xb, sem).wait()

        def copy_ids(j, ib, off):
            for k in range(0, UNIT, 16):
                ib[pl.ds(k, 16)] = itab[j // PACK, 0, pl.ds(off + k, 16)]

        def scatter_start(xb, ib, ssem):
            pltpu.async_copy(xb, acc.at[ib], ssem, add=True)

        def scatter_wait(xb, ib, ssem):
            pltpu.make_async_copy(xb, acc.at[ib], ssem).wait()

        for k in range(DEPTH):
            issue(k, xbs[k], lsems[k])
        icp.wait()

        @pl.when(w < EXTRA)
        def _():
            pltpu.async_copy(b3_hbm.at[pl.ds(u0 // PACK + BASE // PACK, 1)],
                             itab.at[pl.ds(BASE // PACK, 1)], isem).wait()

        @pl.loop(0, cnt // DEPTH)
        def _(p):
            j0 = DEPTH * p
            for k in range(DEPTH):
                wait_load(xbs[k], lsems[k])
                copy_ids(j0 + k, ibs[k], (k % PACK) * UNIT)
                scatter_start(xbs[k], ibs[k], ssems[k])
            for k in range(DEPTH):
                scatter_wait(xbs[k], ibs[k], ssems[k])

                def _reissue(k=k):
                    issue(j0 + DEPTH + k, xbs[k], lsems[k])

                pl.when(j0 + DEPTH + k < cnt)(_reissue)

        t0 = (cnt // DEPTH) * DEPTH
        for k in range(DEPTH - 1):

            def _tail(k=k):
                wait_load(xbs[k], lsems[k])
                copy_ids(t0 + k, ibs[k], (k % PACK) * UNIT)
                scatter_start(xbs[k], ibs[k], ssems[k])

            pl.when(t0 + k < cnt)(_tail)
        for k in range(DEPTH - 1):

            def _tail_wait(k=k):
                scatter_wait(xbs[k], ibs[k], ssems[k])

            pl.when(t0 + k < cnt)(_tail_wait)

        plsc.subcore_barrier()

        pltpu.sync_copy(acc.at[pl.ds(seg0, SEG_COPY)],
                        out_hbm.at[c, pl.ds(seg0, SEG_COPY)])

    return run(x, batch)


def _combine_body(p_ref, o_ref):
    o_ref[...] = p_ref[0] + p_ref[1]


def _tc_combine(partial):
    blk = 2000
    return pl.pallas_call(
        _combine_body,
        grid=(N_SEG // blk,),
        in_specs=[pl.BlockSpec((NC, blk, D), lambda i: (0, i, 0))],
        out_specs=pl.BlockSpec((blk, D), lambda i: (i, 0)),
        out_shape=jax.ShapeDtypeStruct((N_SEG, D), jnp.float32),
    )(partial)


def kernel(x, batch):
    batch3d = batch.astype(jnp.int32).reshape(N_UNITS // PACK, 1, 128)
    partial = _sc_partial(x, batch3d)
    return _tc_combine(partial)
